# initial kernel scaffold (unmeasured)
import jax
import jax.numpy as jnp
from jax import lax
from jax.experimental import pallas as pl
from jax.experimental.pallas import tpu as pltpu


def kernel(
    x,
):
    def body(*refs):
        pass

    out_shape = jax.ShapeDtypeStruct(..., jnp.float32)
    return pl.pallas_call(body, out_shape=out_shape)(...)


# baseline (device time: 759248 ns/iter reference)
import jax
import jax.numpy as jnp
from jax import lax
from jax.experimental import pallas as pl
from jax.experimental.pallas import tpu as pltpu

N_DEV = 16


def kernel(x):
    m_per, n = x.shape

    def body(x_ref, out_ref, send_sems, recv_sems, copy_sem):
        my = lax.axis_index("i")
        left = lax.rem(my - 1 + N_DEV, N_DEV)
        right = lax.rem(my + 1, N_DEV)

        barrier_sem = pltpu.get_barrier_semaphore()
        for nbr in (left, right):
            pl.semaphore_signal(
                barrier_sem, inc=1,
                device_id=(nbr,), device_id_type=pl.DeviceIdType.MESH,
            )
        pl.semaphore_wait(barrier_sem, 2)

        cp = pltpu.make_async_copy(
            x_ref, out_ref.at[pl.ds(my * m_per, m_per), :], copy_sem
        )
        cp.start()
        cp.wait()

        for h in range(N_DEV - 1):
            c = lax.rem(my - h + N_DEV, N_DEV)
            rdma = pltpu.make_async_remote_copy(
                src_ref=out_ref.at[pl.ds(c * m_per, m_per), :],
                dst_ref=out_ref.at[pl.ds(c * m_per, m_per), :],
                send_sem=send_sems.at[h],
                recv_sem=recv_sems.at[h],
                device_id=(right,),
                device_id_type=pl.DeviceIdType.MESH,
            )
            rdma.start()
            rdma.wait()

    return pl.pallas_call(
        body,
        out_shape=jax.ShapeDtypeStruct((N_DEV * m_per, n), x.dtype),
        in_specs=[pl.BlockSpec(memory_space=pltpu.VMEM)],
        out_specs=pl.BlockSpec(memory_space=pl.ANY),
        scratch_shapes=[
            pltpu.SemaphoreType.DMA((N_DEV - 1,)),
            pltpu.SemaphoreType.DMA((N_DEV - 1,)),
            pltpu.SemaphoreType.DMA,
        ],
        compiler_params=pltpu.CompilerParams(
            collective_id=0,
            vmem_limit_bytes=100 * 1024 * 1024,
        ),
    )(x)


# device time: 441226 ns/iter; 1.7208x vs baseline; 1.7208x over previous
import jax
import jax.numpy as jnp
from jax import lax
from jax.experimental import pallas as pl
from jax.experimental.pallas import tpu as pltpu

N_DEV = 16


def kernel(x):
    m_per, n = x.shape
    half = m_per // 2

    def body(x_ref, out_ref, fs_sems, fr_sems, rs_sems, rr_sems, copy_sem):
        my = lax.axis_index("i")
        left = lax.rem(my - 1 + N_DEV, N_DEV)
        right = lax.rem(my + 1, N_DEV)

        barrier_sem = pltpu.get_barrier_semaphore()
        for nbr in (left, right):
            pl.semaphore_signal(
                barrier_sem, inc=1,
                device_id=(nbr,), device_id_type=pl.DeviceIdType.MESH,
            )
        pl.semaphore_wait(barrier_sem, 2)

        cp = pltpu.make_async_copy(
            x_ref, out_ref.at[pl.ds(my * m_per, m_per), :], copy_sem
        )
        cp.start()
        cp.wait()

        for h in range(N_DEV - 1):
            cf = lax.rem(my - h + N_DEV, N_DEV)
            fwd = pltpu.make_async_remote_copy(
                src_ref=out_ref.at[pl.ds(cf * m_per, half), :],
                dst_ref=out_ref.at[pl.ds(cf * m_per, half), :],
                send_sem=fs_sems.at[h],
                recv_sem=fr_sems.at[h],
                device_id=(right,),
                device_id_type=pl.DeviceIdType.MESH,
            )
            cr = lax.rem(my + h, N_DEV)
            rev = pltpu.make_async_remote_copy(
                src_ref=out_ref.at[pl.ds(cr * m_per + half, half), :],
                dst_ref=out_ref.at[pl.ds(cr * m_per + half, half), :],
                send_sem=rs_sems.at[h],
                recv_sem=rr_sems.at[h],
                device_id=(left,),
                device_id_type=pl.DeviceIdType.MESH,
            )
            fwd.start()
            rev.start()
            fwd.wait()
            rev.wait()

    return pl.pallas_call(
        body,
        out_shape=jax.ShapeDtypeStruct((N_DEV * m_per, n), x.dtype),
        in_specs=[pl.BlockSpec(memory_space=pltpu.VMEM)],
        out_specs=pl.BlockSpec(memory_space=pl.ANY),
        scratch_shapes=[
            pltpu.SemaphoreType.DMA((N_DEV - 1,)),
            pltpu.SemaphoreType.DMA((N_DEV - 1,)),
            pltpu.SemaphoreType.DMA((N_DEV - 1,)),
            pltpu.SemaphoreType.DMA((N_DEV - 1,)),
            pltpu.SemaphoreType.DMA,
        ],
        compiler_params=pltpu.CompilerParams(
            collective_id=0,
            vmem_limit_bytes=100 * 1024 * 1024,
        ),
    )(x)


# device time: 392760 ns/iter; 1.9331x vs baseline; 1.1234x over previous
import jax
import jax.numpy as jnp
from jax import lax
from jax.experimental import pallas as pl
from jax.experimental.pallas import tpu as pltpu

N_DEV = 16
NHOP = N_DEV - 1
NPIECE = 2


def kernel(x):
    m_per, n = x.shape
    half = m_per // 2
    piece = half // NPIECE

    def body(x_ref, out_ref, fs_sems, fr_sems, rs_sems, rr_sems, copy_sem):
        my = lax.axis_index("i")
        left = lax.rem(my - 1 + N_DEV, N_DEV)
        right = lax.rem(my + 1, N_DEV)

        barrier_sem = pltpu.get_barrier_semaphore()
        for nbr in (left, right):
            pl.semaphore_signal(
                barrier_sem, inc=1,
                device_id=(nbr,), device_id_type=pl.DeviceIdType.MESH,
            )
        pl.semaphore_wait(barrier_sem, 2)

        cp = pltpu.make_async_copy(
            x_ref, out_ref.at[pl.ds(my * m_per, m_per), :], copy_sem
        )
        cp.start()
        cp.wait()

        def fwd_rdma(h, p):
            c = lax.rem(my - h + N_DEV, N_DEV)
            off = c * m_per + p * piece
            return pltpu.make_async_remote_copy(
                src_ref=out_ref.at[pl.ds(off, piece), :],
                dst_ref=out_ref.at[pl.ds(off, piece), :],
                send_sem=fs_sems.at[h, p],
                recv_sem=fr_sems.at[h, p],
                device_id=(right,),
                device_id_type=pl.DeviceIdType.MESH,
            )

        def rev_rdma(h, p):
            c = lax.rem(my + h, N_DEV)
            off = c * m_per + half + p * piece
            return pltpu.make_async_remote_copy(
                src_ref=out_ref.at[pl.ds(off, piece), :],
                dst_ref=out_ref.at[pl.ds(off, piece), :],
                send_sem=rs_sems.at[h, p],
                recv_sem=rr_sems.at[h, p],
                device_id=(left,),
                device_id_type=pl.DeviceIdType.MESH,
            )

        for p in range(NPIECE):
            fwd_rdma(0, p).start()
            rev_rdma(0, p).start()

        for h in range(1, NHOP):
            for p in range(NPIECE):
                fwd_rdma(h - 1, p).wait_recv()
                fwd_rdma(h, p).start()
                rev_rdma(h - 1, p).wait_recv()
                rev_rdma(h, p).start()

        for p in range(NPIECE):
            fwd_rdma(NHOP - 1, p).wait_recv()
            rev_rdma(NHOP - 1, p).wait_recv()
        for h in range(NHOP):
            for p in range(NPIECE):
                fwd_rdma(h, p).wait_send()
                rev_rdma(h, p).wait_send()

    return pl.pallas_call(
        body,
        out_shape=jax.ShapeDtypeStruct((N_DEV * m_per, n), x.dtype),
        in_specs=[pl.BlockSpec(memory_space=pltpu.VMEM)],
        out_specs=pl.BlockSpec(memory_space=pl.ANY),
        scratch_shapes=[
            pltpu.SemaphoreType.DMA((NHOP, NPIECE)),
            pltpu.SemaphoreType.DMA((NHOP, NPIECE)),
            pltpu.SemaphoreType.DMA((NHOP, NPIECE)),
            pltpu.SemaphoreType.DMA((NHOP, NPIECE)),
            pltpu.SemaphoreType.DMA,
        ],
        compiler_params=pltpu.CompilerParams(
            collective_id=0,
            vmem_limit_bytes=100 * 1024 * 1024,
        ),
    )(x)
